# baseline (speedup 1.0000x reference)
import functools

import jax
import jax.numpy as jnp
from jax import lax
from jax.experimental import pallas as pl
from jax.experimental.pallas import tpu as pltpu
from jax.experimental.pallas import tpu_sc as plsc

BINS = 10
EPS = 1e-08
ALPHA = 1.0 / (2 * BINS)
EDGES = [float(x) / BINS for x in range(BINS + 1)]
LOS = [EDGES[i] - ALPHA for i in range(BINS + 1)]

ROWS_PER_BLOCK = 1024
NS = 16
LANES = 16
GCH = 128


def _stage1_body(pred_ref, s_ref, lsp_ref):
    x = pred_ref[...]
    s = jnp.sum(jnp.exp(x), axis=1)
    s_ref[...] = s
    lsp_ref[...] = jnp.log(s + EPS)


def _make_sc_stage2(batch, ncols):
    ch = batch // NS
    nv = ch // LANES
    ng = ch // GCH

    def dg(vec, idx):
        dn = lax.GatherDimensionNumbers(
            offset_dims=(), collapsed_slice_dims=(0,), start_index_map=(0,))
        return lax.gather(vec, idx[:, None], dn, slice_sizes=(1,),
                          mode=lax.GatherScatterMode.PROMISE_IN_BOUNDS)

    def body(tgt_hbm, predflat_hbm, s_hbm, lsp_hbm, out_hbm,
             t_v, idx_v, p_v, s_v, lsp_v, cnt_all_v, acc_all_v,
             stage_cnt_v, stage_acc_v, stage_out_v, sh_cnt, sh_acc, sem):
        cid = lax.axis_index("c")
        sid = lax.axis_index("s")
        base = sid * ch
        lane = lax.broadcasted_iota(jnp.int32, (LANES,), 0)

        def lane_sum(x):
            for sh in (8, 4, 2, 1):
                x = x + dg(x, (lane + sh) % LANES)
            return x

        pltpu.sync_copy(tgt_hbm.at[pl.ds(base, ch)], t_v)
        pltpu.sync_copy(s_hbm.at[pl.ds(base, ch)], s_v)
        pltpu.sync_copy(lsp_hbm.at[pl.ds(base, ch)], lsp_v)

        def idx_body(j, carry):
            t = t_v[pl.ds(j * LANES, LANES)]
            rows = base + j * LANES + lane
            idx_v[pl.ds(j * LANES, LANES)] = rows * ncols + t
            return carry

        lax.fori_loop(0, nv, idx_body, 0)

        copies = []
        for k in range(ng):
            copies.append(pltpu.async_copy(
                predflat_hbm.at[idx_v.at[pl.ds(k * GCH, GCH)]],
                p_v.at[pl.ds(k * GCH, GCH)], sem))
        for c in copies:
            c.wait()

        def g_of(j):
            p = p_v[pl.ds(j * LANES, LANES)]
            s = s_v[pl.ds(j * LANES, LANES)]
            return p, 1.0 - jnp.exp(p) / s

        def cnt_body(j, cnts):
            _, g = g_of(j)
            ge = [g >= LOS[b] for b in range(BINS + 1)]
            ms = [ge[b] & (~ge[b + 1]) for b in range(BINS)] + [ge[BINS]]
            return tuple(cnts[b] + ms[b].astype(jnp.int32)
                         for b in range(BINS + 1))

        cnts = lax.fori_loop(
            0, nv, cnt_body,
            tuple(jnp.zeros((LANES,), jnp.int32) for _ in range(BINS + 1)))

        counts = jnp.zeros((LANES,), jnp.int32)
        for b in range(BINS + 1):
            counts = jnp.where(lane == b, lane_sum(cnts[b]), counts)

        stage_cnt_v[...] = counts
        pltpu.sync_copy(stage_cnt_v, sh_cnt.at[pl.ds(sid * LANES, LANES)])
        plsc.subcore_barrier()
        pltpu.sync_copy(sh_cnt, cnt_all_v)

        cg = cnt_all_v[pl.ds(0, LANES)]
        for k in range(1, NS):
            cg = cg + cnt_all_v[pl.ds(k * LANES, LANES)]

        nonempty = (cg > 0) & (lane < BINS + 1)
        nf = lane_sum(nonempty.astype(jnp.int32)).astype(jnp.float32)
        inv_all = jnp.where(
            cg > 0,
            (1.0 / jnp.maximum(cg, 1).astype(jnp.float32)) / nf,
            jnp.float32(0.0),
        )

        def acc_body(j, acc):
            p, g = g_of(j)
            lsp = lsp_v[pl.ds(j * LANES, LANES)]
            bi = jnp.zeros((LANES,), jnp.int32)
            for b in range(BINS + 1):
                bi = bi + (g >= LOS[b]).astype(jnp.int32)
            bi = jnp.maximum(bi - 1, 0)
            w = dg(inv_all, bi)
            return acc + (lsp - p) * w

        acc = lax.fori_loop(0, nv, acc_body, jnp.zeros((LANES,), jnp.float32))

        stage_acc_v[...] = acc
        pltpu.sync_copy(stage_acc_v, sh_acc.at[pl.ds(sid * LANES, LANES)])
        plsc.subcore_barrier()

        @pl.when(jnp.logical_and(cid == 0, sid == 0))
        def _():
            pltpu.sync_copy(sh_acc, acc_all_v)
            tot = acc_all_v[pl.ds(0, LANES)]
            for k in range(1, NS):
                tot = tot + acc_all_v[pl.ds(k * LANES, LANES)]
            total = lane_sum(tot)
            stage_out_v[...] = jnp.where(lane == 0, total, jnp.float32(0.0))
            pltpu.sync_copy(stage_out_v, out_hbm)

    mesh = plsc.VectorSubcoreMesh(core_axis_name="c", subcore_axis_name="s")
    return pl.kernel(
        body,
        out_type=jax.ShapeDtypeStruct((LANES,), jnp.float32),
        mesh=mesh,
        compiler_params=pltpu.CompilerParams(needs_layout_passes=False),
        scratch_types=[
            pltpu.VMEM((ch,), jnp.int32),
            pltpu.VMEM((ch,), jnp.int32),
            pltpu.VMEM((ch,), jnp.float32),
            pltpu.VMEM((ch,), jnp.float32),
            pltpu.VMEM((ch,), jnp.float32),
            pltpu.VMEM((NS * LANES,), jnp.int32),
            pltpu.VMEM((NS * LANES,), jnp.float32),
            pltpu.VMEM((LANES,), jnp.int32),
            pltpu.VMEM((LANES,), jnp.float32),
            pltpu.VMEM((LANES,), jnp.float32),
            pltpu.VMEM_SHARED((NS * LANES,), jnp.int32),
            pltpu.VMEM_SHARED((NS * LANES,), jnp.float32),
            pltpu.SemaphoreType.DMA,
        ],
    )


def kernel(pred, target):
    batch, ncols = pred.shape
    target = target.astype(jnp.int32)
    nblocks = batch // ROWS_PER_BLOCK

    s, lsp = pl.pallas_call(
        _stage1_body,
        grid=(nblocks,),
        in_specs=[pl.BlockSpec((ROWS_PER_BLOCK, ncols), lambda i: (i, 0))],
        out_specs=[
            pl.BlockSpec((ROWS_PER_BLOCK,), lambda i: (i,)),
            pl.BlockSpec((ROWS_PER_BLOCK,), lambda i: (i,)),
        ],
        out_shape=[
            jax.ShapeDtypeStruct((batch,), jnp.float32),
            jax.ShapeDtypeStruct((batch,), jnp.float32),
        ],
    )(pred)

    out = _make_sc_stage2(batch, ncols)(target, pred.reshape(-1), s, lsp)
    return out[0]


# speedup vs baseline: 1.6711x; 1.6711x over previous
import jax
import jax.numpy as jnp
from jax import lax
from jax.experimental import pallas as pl
from jax.experimental.pallas import tpu as pltpu
from jax.experimental.pallas import tpu_sc as plsc

BINS = 10
EPS = 1e-08
ALPHA = 1.0 / (2 * BINS)
EDGES = [float(x) / BINS for x in range(BINS + 1)]
LOS = [EDGES[i] - ALPHA for i in range(BINS + 1)]

ROWS_PER_BLOCK = 1024
NS = 16
LANES = 16


def _stage1_body(pred_ref, tgt_ref, g_ref, loss_ref):
    x = pred_ref[...]
    t = tgt_ref[...]
    s = jnp.sum(jnp.exp(x), axis=1)
    cols = lax.broadcasted_iota(jnp.int32, x.shape, 1)
    p = jnp.sum(jnp.where(cols == t[:, None], x, 0.0), axis=1)
    g_ref[...] = 1.0 - jnp.exp(p) / s
    loss_ref[...] = -p + jnp.log(s + EPS)


def _make_sc_stage2(batch):
    ch = batch // NS
    nv = ch // LANES

    def dg(vec, idx):
        dn = lax.GatherDimensionNumbers(
            offset_dims=(), collapsed_slice_dims=(0,), start_index_map=(0,))
        return lax.gather(vec, idx[:, None], dn, slice_sizes=(1,),
                          mode=lax.GatherScatterMode.PROMISE_IN_BOUNDS)

    def body(g_hbm, loss_hbm, out_hbm,
             g_v, loss_v, cnt_all_v, acc_all_v,
             stage_cnt_v, stage_acc_v, stage_out_v, sh_cnt, sh_acc):
        cid = lax.axis_index("c")
        sid = lax.axis_index("s")
        base = sid * ch
        lane = lax.broadcasted_iota(jnp.int32, (LANES,), 0)

        def lane_sum(x):
            for sh in (8, 4, 2, 1):
                x = x + dg(x, (lane + sh) % LANES)
            return x

        pltpu.sync_copy(g_hbm.at[pl.ds(base, ch)], g_v)
        pltpu.sync_copy(loss_hbm.at[pl.ds(base, ch)], loss_v)

        def cnt_body(j, cnts):
            g = g_v[pl.ds(j * LANES, LANES)]
            ge = [g >= LOS[b] for b in range(BINS + 1)]
            ms = [ge[b] & (~ge[b + 1]) for b in range(BINS)] + [ge[BINS]]
            return tuple(cnts[b] + ms[b].astype(jnp.int32)
                         for b in range(BINS + 1))

        cnts = lax.fori_loop(
            0, nv, cnt_body,
            tuple(jnp.zeros((LANES,), jnp.int32) for _ in range(BINS + 1)))

        counts = jnp.zeros((LANES,), jnp.int32)
        for b in range(BINS + 1):
            counts = jnp.where(lane == b, lane_sum(cnts[b]), counts)

        stage_cnt_v[...] = counts
        pltpu.sync_copy(stage_cnt_v, sh_cnt.at[pl.ds(sid * LANES, LANES)])
        plsc.subcore_barrier()
        pltpu.sync_copy(sh_cnt, cnt_all_v)

        cg = cnt_all_v[pl.ds(0, LANES)]
        for k in range(1, NS):
            cg = cg + cnt_all_v[pl.ds(k * LANES, LANES)]

        nonempty = (cg > 0) & (lane < BINS + 1)
        nf = lane_sum(nonempty.astype(jnp.int32)).astype(jnp.float32)
        inv_all = jnp.where(
            cg > 0,
            (1.0 / jnp.maximum(cg, 1).astype(jnp.float32)) / nf,
            jnp.float32(0.0),
        )

        def acc_body(j, acc):
            g = g_v[pl.ds(j * LANES, LANES)]
            loss = loss_v[pl.ds(j * LANES, LANES)]
            bi = jnp.zeros((LANES,), jnp.int32)
            for b in range(BINS + 1):
                bi = bi + (g >= LOS[b]).astype(jnp.int32)
            bi = jnp.maximum(bi - 1, 0)
            return acc + loss * dg(inv_all, bi)

        acc = lax.fori_loop(0, nv, acc_body, jnp.zeros((LANES,), jnp.float32))

        stage_acc_v[...] = acc
        pltpu.sync_copy(stage_acc_v, sh_acc.at[pl.ds(sid * LANES, LANES)])
        plsc.subcore_barrier()

        @pl.when(jnp.logical_and(cid == 0, sid == 0))
        def _():
            pltpu.sync_copy(sh_acc, acc_all_v)
            tot = acc_all_v[pl.ds(0, LANES)]
            for k in range(1, NS):
                tot = tot + acc_all_v[pl.ds(k * LANES, LANES)]
            total = lane_sum(tot)
            stage_out_v[...] = jnp.where(lane == 0, total, jnp.float32(0.0))
            pltpu.sync_copy(stage_out_v, out_hbm)

    mesh = plsc.VectorSubcoreMesh(core_axis_name="c", subcore_axis_name="s")
    return pl.kernel(
        body,
        out_type=jax.ShapeDtypeStruct((LANES,), jnp.float32),
        mesh=mesh,
        compiler_params=pltpu.CompilerParams(needs_layout_passes=False),
        scratch_types=[
            pltpu.VMEM((ch,), jnp.float32),
            pltpu.VMEM((ch,), jnp.float32),
            pltpu.VMEM((NS * LANES,), jnp.int32),
            pltpu.VMEM((NS * LANES,), jnp.float32),
            pltpu.VMEM((LANES,), jnp.int32),
            pltpu.VMEM((LANES,), jnp.float32),
            pltpu.VMEM((LANES,), jnp.float32),
            pltpu.VMEM_SHARED((NS * LANES,), jnp.int32),
            pltpu.VMEM_SHARED((NS * LANES,), jnp.float32),
        ],
    )


def kernel(pred, target):
    batch, ncols = pred.shape
    target = target.astype(jnp.int32)
    nblocks = batch // ROWS_PER_BLOCK

    g, loss = pl.pallas_call(
        _stage1_body,
        grid=(nblocks,),
        in_specs=[
            pl.BlockSpec((ROWS_PER_BLOCK, ncols), lambda i: (i, 0)),
            pl.BlockSpec((ROWS_PER_BLOCK,), lambda i: (i,)),
        ],
        out_specs=[
            pl.BlockSpec((ROWS_PER_BLOCK,), lambda i: (i,)),
            pl.BlockSpec((ROWS_PER_BLOCK,), lambda i: (i,)),
        ],
        out_shape=[
            jax.ShapeDtypeStruct((batch,), jnp.float32),
            jax.ShapeDtypeStruct((batch,), jnp.float32),
        ],
    )(pred, target)

    out = _make_sc_stage2(batch)(g, loss)
    return out[0]


# speedup vs baseline: 4.2579x; 2.5480x over previous
import jax
import jax.numpy as jnp
from jax import lax
from jax.experimental import pallas as pl
from jax.experimental.pallas import tpu as pltpu
from jax.experimental.pallas import tpu_sc as plsc

BINS = 10
EPS = 1e-08
ALPHA = 1.0 / (2 * BINS)
EDGES = [float(x) / BINS for x in range(BINS + 1)]
LOS = [EDGES[i] - ALPHA for i in range(BINS + 1)]

BATCH_PER_BLOCK = 2048
NS = 16
LANES = 16


def _stage1_body(predt_ref, tgt_ref, g_ref, loss_ref):
    x = predt_ref[...]
    t = tgt_ref[...]
    s = jnp.sum(jnp.exp(x), axis=0)
    rows = lax.broadcasted_iota(jnp.int32, x.shape, 0)
    p = jnp.sum(jnp.where(rows == t[None, :], x, 0.0), axis=0)
    g_ref[...] = 1.0 - jnp.exp(p) / s
    loss_ref[...] = -p + jnp.log(s + EPS)


def _make_sc_stage2(batch):
    ch = batch // NS
    nv = ch // LANES

    def dg(vec, idx):
        dn = lax.GatherDimensionNumbers(
            offset_dims=(), collapsed_slice_dims=(0,), start_index_map=(0,))
        return lax.gather(vec, idx[:, None], dn, slice_sizes=(1,),
                          mode=lax.GatherScatterMode.PROMISE_IN_BOUNDS)

    def body(g_hbm, loss_hbm, out_hbm,
             g_v, loss_v, cnt_all_v, acc_all_v,
             stage_cnt_v, stage_acc_v, stage_out_v, sh_cnt, sh_acc):
        cid = lax.axis_index("c")
        sid = lax.axis_index("s")
        base = sid * ch
        lane = lax.broadcasted_iota(jnp.int32, (LANES,), 0)

        def lane_sum(x):
            for sh in (8, 4, 2, 1):
                x = x + dg(x, (lane + sh) % LANES)
            return x

        pltpu.sync_copy(g_hbm.at[pl.ds(base, ch)], g_v)
        pltpu.sync_copy(loss_hbm.at[pl.ds(base, ch)], loss_v)

        def cnt_body(j, cnts):
            g = g_v[pl.ds(j * LANES, LANES)]
            ge = [g >= LOS[b] for b in range(BINS + 1)]
            ms = [ge[b] & (~ge[b + 1]) for b in range(BINS)] + [ge[BINS]]
            return tuple(cnts[b] + ms[b].astype(jnp.int32)
                         for b in range(BINS + 1))

        cnts = lax.fori_loop(
            0, nv, cnt_body,
            tuple(jnp.zeros((LANES,), jnp.int32) for _ in range(BINS + 1)))

        counts = jnp.zeros((LANES,), jnp.int32)
        for b in range(BINS + 1):
            counts = jnp.where(lane == b, lane_sum(cnts[b]), counts)

        stage_cnt_v[...] = counts
        pltpu.sync_copy(stage_cnt_v, sh_cnt.at[pl.ds(sid * LANES, LANES)])
        plsc.subcore_barrier()
        pltpu.sync_copy(sh_cnt, cnt_all_v)

        cg = cnt_all_v[pl.ds(0, LANES)]
        for k in range(1, NS):
            cg = cg + cnt_all_v[pl.ds(k * LANES, LANES)]

        nonempty = (cg > 0) & (lane < BINS + 1)
        nf = lane_sum(nonempty.astype(jnp.int32)).astype(jnp.float32)
        inv_all = jnp.where(
            cg > 0,
            (1.0 / jnp.maximum(cg, 1).astype(jnp.float32)) / nf,
            jnp.float32(0.0),
        )

        def acc_body(j, acc):
            g = g_v[pl.ds(j * LANES, LANES)]
            loss = loss_v[pl.ds(j * LANES, LANES)]
            bi = jnp.zeros((LANES,), jnp.int32)
            for b in range(BINS + 1):
                bi = bi + (g >= LOS[b]).astype(jnp.int32)
            bi = jnp.maximum(bi - 1, 0)
            return acc + loss * dg(inv_all, bi)

        acc = lax.fori_loop(0, nv, acc_body, jnp.zeros((LANES,), jnp.float32))

        stage_acc_v[...] = acc
        pltpu.sync_copy(stage_acc_v, sh_acc.at[pl.ds(sid * LANES, LANES)])
        plsc.subcore_barrier()

        @pl.when(jnp.logical_and(cid == 0, sid == 0))
        def _():
            pltpu.sync_copy(sh_acc, acc_all_v)
            tot = acc_all_v[pl.ds(0, LANES)]
            for k in range(1, NS):
                tot = tot + acc_all_v[pl.ds(k * LANES, LANES)]
            total = lane_sum(tot)
            stage_out_v[...] = jnp.where(lane == 0, total, jnp.float32(0.0))
            pltpu.sync_copy(stage_out_v, out_hbm)

    mesh = plsc.VectorSubcoreMesh(core_axis_name="c", subcore_axis_name="s")
    return pl.kernel(
        body,
        out_type=jax.ShapeDtypeStruct((LANES,), jnp.float32),
        mesh=mesh,
        compiler_params=pltpu.CompilerParams(needs_layout_passes=False),
        scratch_types=[
            pltpu.VMEM((ch,), jnp.float32),
            pltpu.VMEM((ch,), jnp.float32),
            pltpu.VMEM((NS * LANES,), jnp.int32),
            pltpu.VMEM((NS * LANES,), jnp.float32),
            pltpu.VMEM((LANES,), jnp.int32),
            pltpu.VMEM((LANES,), jnp.float32),
            pltpu.VMEM((LANES,), jnp.float32),
            pltpu.VMEM_SHARED((NS * LANES,), jnp.int32),
            pltpu.VMEM_SHARED((NS * LANES,), jnp.float32),
        ],
    )


def kernel(pred, target):
    batch, ncols = pred.shape
    target = target.astype(jnp.int32)
    nblocks = batch // BATCH_PER_BLOCK

    g, loss = pl.pallas_call(
        _stage1_body,
        grid=(nblocks,),
        in_specs=[
            pl.BlockSpec((ncols, BATCH_PER_BLOCK), lambda i: (0, i)),
            pl.BlockSpec((BATCH_PER_BLOCK,), lambda i: (i,)),
        ],
        out_specs=[
            pl.BlockSpec((BATCH_PER_BLOCK,), lambda i: (i,)),
            pl.BlockSpec((BATCH_PER_BLOCK,), lambda i: (i,)),
        ],
        out_shape=[
            jax.ShapeDtypeStruct((batch,), jnp.float32),
            jax.ShapeDtypeStruct((batch,), jnp.float32),
        ],
    )(pred.T, target)

    out = _make_sc_stage2(batch)(g, loss)
    return out[0]
